# baseline (speedup 1.0000x reference)
import jax
import jax.numpy as jnp
from jax import lax
from jax.experimental import pallas as pl
from jax.experimental.pallas import tpu as pltpu
from jax.experimental.pallas import tpu_sc as plsc

D = 32
HIST = 200
N_ITEMS = 50
GH = 1000
EP_LEN = 200

ITEM_PAD = 64
HIST_PAD = 216
GH_PAD = 1008
ROWS_PER_TILE = 8
N_HTILES = HIST // ROWS_PER_TILE

OUT_LEN = N_ITEMS * D + HIST * D + D + EP_LEN
SACH_OFF = N_ITEMS * D
SUC_OFF = SACH_OFF + HIST * D
PRED_OFF = SUC_OFF + D

NEG_BIG = -1e30


def _body(uidx_h, cidx_h, iidx_h, hidx_h, gh_h, rat_h, noi_h, preds_h,
          utab_h, rtab_h, ctab_h, out_h,
          uidx_v, cidx_v, iidx_v, hidx16_v, gh_v, rat_v, rat16_v, noi16_v,
          item_v, itemT_v, hrow_v, srow_v, urow_v, crow_v,
          sach_v, sui_v, suc_v, pred_v,
          sem_a, sem_b, sem_c):
    c = lax.axis_index("c")
    s = lax.axis_index("s")
    wid = s * 2 + c

    base = pl.multiple_of(jnp.minimum(wid * ROWS_PER_TILE, HIST), 8)
    sui_off = pl.multiple_of(jnp.clip((wid - N_HTILES) * 16, 0, 48), 8)

    cp_i = pltpu.async_copy(iidx_h, iidx_v, sem_a)
    cp_h = pltpu.async_copy(hidx_h.at[pl.ds(base, 16)], hidx16_v, sem_a)
    cp_g = pltpu.async_copy(gh_h, gh_v, sem_b)
    cp_r = pltpu.async_copy(rat_h, rat_v, sem_b)
    cp_r16 = pltpu.async_copy(rat_h.at[pl.ds(base, 16)], rat16_v, sem_b)
    cp_n16 = pltpu.async_copy(noi_h.at[pl.ds(base, 16)], noi16_v, sem_b)
    cp_u = pltpu.async_copy(uidx_h, uidx_v, sem_b)
    cp_c = pltpu.async_copy(cidx_h, cidx_v, sem_b)
    cp_i.wait()
    cp_h.wait()

    g_item = pltpu.async_copy(rtab_h.at[iidx_v], item_v, sem_c)
    g_hist = pltpu.async_copy(rtab_h.at[hidx16_v.at[pl.ds(0, ROWS_PER_TILE)]],
                              hrow_v, sem_c)
    g_sui = pltpu.async_copy(rtab_h.at[iidx_v.at[pl.ds(sui_off, 16)]],
                             srow_v, sem_c)

    cp_g.wait()
    cp_r.wait()
    cp_r16.wait()
    cp_n16.wait()
    cp_u.wait()
    cp_c.wait()
    g_user = pltpu.async_copy(utab_h.at[uidx_v], urow_v, sem_a)
    g_cat = pltpu.async_copy(ctab_h.at[cidx_v], crow_v, sem_a)

    s1 = jnp.zeros((16,), jnp.float32)
    s2 = jnp.zeros((16,), jnp.float32)
    for b in range(13):
        rf = rat_v[pl.ds(b * 16, 16)].astype(jnp.float32)
        s1 = s1 + rf
        s2 = s2 + rf * rf
    S1 = jnp.sum(s1)
    S2 = jnp.sum(s2)
    r_hist = jnp.float32(1.0 / HIST)
    rmean = S1 * r_hist
    rvar = (S2 - S1 * S1 * r_hist) * jnp.float32(1.0 / (HIST - 1))

    g_item.wait()
    g_hist.wait()
    g_sui.wait()

    d_lo = lax.iota(jnp.int32, 16)
    d_hi = d_lo + 16
    for j in range(ITEM_PAD):
        jv = jnp.full((16,), j, jnp.int32)
        plsc.store_scatter(itemT_v, [d_lo, jv], item_v[j, pl.ds(0, 16)])
        plsc.store_scatter(itemT_v, [d_hi, jv], item_v[j, pl.ds(16, 16)])

    lane_ok = [(lax.iota(jnp.int32, 16) + 16 * k) < N_ITEMS for k in range(4)]

    @pl.when(wid < N_HTILES)
    def _():
        ratw = rat16_v[pl.ds(0, 16)].astype(jnp.float32)
        noiw = noi16_v[pl.ds(0, 16)]
        hidw = hidx16_v[pl.ds(0, 16)]
        for hh in range(ROWS_PER_TILE):
            mask_s = ((5.0 - ratw[hh]) * 0.2
                      + (rmean * 0.2 + rvar * noiw[hh]) * 0.2)
            hid = hidw[hh]
            acc = jnp.zeros((16,), jnp.float32)
            for b in range(GH_PAD // 16):
                acc = acc + jnp.where(gh_v[pl.ds(b * 16, 16)] == hid, 1.0, 0.0)
            cnt = jnp.sum(acc)
            m = mask_s * (1.0 - cnt * 0.1)

            hr_lo = hrow_v[hh, pl.ds(0, 16)] * m
            hr_hi = hrow_v[hh, pl.ds(16, 16)] * m
            l0 = jnp.zeros((16,), jnp.float32)
            l1 = jnp.zeros((16,), jnp.float32)
            l2 = jnp.zeros((16,), jnp.float32)
            l3 = jnp.zeros((16,), jnp.float32)
            for d in range(D):
                sc = hr_lo[d] if d < 16 else hr_hi[d - 16]
                l0 = l0 + sc * itemT_v[d, pl.ds(0, 16)]
                l1 = l1 + sc * itemT_v[d, pl.ds(16, 16)]
                l2 = l2 + sc * itemT_v[d, pl.ds(32, 16)]
                l3 = l3 + sc * itemT_v[d, pl.ds(48, 16)]
            l0 = jnp.where(lane_ok[0], l0, NEG_BIG)
            l1 = jnp.where(lane_ok[1], l1, NEG_BIG)
            l2 = jnp.where(lane_ok[2], l2, NEG_BIG)
            l3 = jnp.where(lane_ok[3], l3, NEG_BIG)
            mx = jnp.max(jnp.maximum(jnp.maximum(l0, l1),
                                     jnp.maximum(l2, l3)))
            es = [jnp.exp(l0 - mx), jnp.exp(l1 - mx),
                  jnp.exp(l2 - mx), jnp.exp(l3 - mx)]
            z = jnp.sum(es[0] + es[1] + es[2] + es[3])
            a_lo = jnp.zeros((16,), jnp.float32)
            a_hi = jnp.zeros((16,), jnp.float32)
            for j in range(N_ITEMS):
                aj = es[j // 16][j % 16]
                a_lo = a_lo + aj * item_v[j, pl.ds(0, 16)]
                a_hi = a_hi + aj * item_v[j, pl.ds(16, 16)]
            sach_v[pl.ds(hh * D, 16)] = a_lo / z
            sach_v[pl.ds(hh * D + 16, 16)] = a_hi / z
        pltpu.sync_copy(
            sach_v,
            out_h.at[pl.ds(SACH_OFF + wid * (ROWS_PER_TILE * D),
                           ROWS_PER_TILE * D)])

    g_user.wait()
    g_cat.wait()

    @pl.when(jnp.logical_and(wid >= N_HTILES, wid <= 28))
    def _():
        u_lo = urow_v[0, pl.ds(0, 16)]
        u_hi = urow_v[0, pl.ds(16, 16)]
        for jj in range(16):
            sui_v[pl.ds(jj * D, 16)] = u_lo * srow_v[jj, pl.ds(0, 16)]
            sui_v[pl.ds(jj * D + 16, 16)] = u_hi * srow_v[jj, pl.ds(16, 16)]

    @pl.when(jnp.logical_and(wid >= N_HTILES, wid <= 27))
    def _():
        pltpu.sync_copy(sui_v, out_h.at[pl.ds((wid - N_HTILES) * (16 * D),
                                              16 * D)])

    @pl.when(wid == 28)
    def _():
        pltpu.sync_copy(sui_v.at[pl.ds(0, 2 * D)],
                        out_h.at[pl.ds(48 * D, 2 * D)])

    @pl.when(wid == 30)
    def _():
        pltpu.sync_copy(preds_h, pred_v)
        pltpu.sync_copy(pred_v, out_h.at[pl.ds(PRED_OFF, EP_LEN)])

    @pl.when(wid == 31)
    def _():
        suc_v[pl.ds(0, 16)] = urow_v[0, pl.ds(0, 16)] * crow_v[0, pl.ds(0, 16)]
        suc_v[pl.ds(16, 16)] = (urow_v[0, pl.ds(16, 16)]
                                * crow_v[0, pl.ds(16, 16)])
        pltpu.sync_copy(suc_v, out_h.at[pl.ds(SUC_OFF, D)])


@jax.jit
def _sc_forward(uidx8, cidx8, iidx64, hidx, gh, rat, noi, preds,
                user_table, recipe_table, category_table):
    mesh = plsc.VectorSubcoreMesh(core_axis_name="c", subcore_axis_name="s")
    f = pl.kernel(
        _body,
        out_type=jax.ShapeDtypeStruct((OUT_LEN,), jnp.float32),
        mesh=mesh,
        compiler_params=pltpu.CompilerParams(needs_layout_passes=False,
                                             use_tc_tiling_on_sc=False),
        scratch_types=[
            pltpu.VMEM((8,), jnp.int32),
            pltpu.VMEM((8,), jnp.int32),
            pltpu.VMEM((ITEM_PAD,), jnp.int32),
            pltpu.VMEM((16,), jnp.int32),
            pltpu.VMEM((GH_PAD,), jnp.int32),
            pltpu.VMEM((HIST_PAD,), jnp.int32),
            pltpu.VMEM((16,), jnp.int32),
            pltpu.VMEM((16,), jnp.float32),
            pltpu.VMEM((ITEM_PAD, D), jnp.float32),
            pltpu.VMEM((D, ITEM_PAD), jnp.float32),
            pltpu.VMEM((ROWS_PER_TILE, D), jnp.float32),
            pltpu.VMEM((16, D), jnp.float32),
            pltpu.VMEM((8, D), jnp.float32),
            pltpu.VMEM((8, D), jnp.float32),
            pltpu.VMEM((ROWS_PER_TILE * D,), jnp.float32),
            pltpu.VMEM((16 * D,), jnp.float32),
            pltpu.VMEM((D,), jnp.float32),
            pltpu.VMEM((EP_LEN,), jnp.float32),
            pltpu.SemaphoreType.DMA,
            pltpu.SemaphoreType.DMA,
            pltpu.SemaphoreType.DMA,
        ],
    )
    return f(uidx8, cidx8, iidx64, hidx, gh, rat, noi, preds,
             user_table, recipe_table, category_table)


def kernel(user_ids, item_id, idx, history, global_history, rating, preds,
           last_category, repetition, user_table, recipe_table,
           category_table):
    i32 = jnp.int32
    uid = jnp.asarray(user_ids, i32)
    lc = jnp.asarray(last_category, i32) - 1
    uidx8 = jnp.full((8,), uid, i32)
    cidx8 = jnp.full((8,), lc, i32)
    iidx64 = jnp.concatenate(
        [item_id.astype(i32), jnp.zeros((ITEM_PAD - N_ITEMS,), i32)])
    hidx = jnp.concatenate(
        [history.astype(i32), jnp.zeros((HIST_PAD - HIST,), i32)])
    gh = jnp.concatenate(
        [global_history.astype(i32), jnp.full((GH_PAD - GH,), -1, i32)])
    rat = jnp.concatenate(
        [rating.astype(i32), jnp.zeros((HIST_PAD - HIST,), i32)])
    noise = jax.random.normal(jax.random.key(42), (HIST,), dtype=jnp.float32)
    noi = jnp.concatenate([noise, jnp.zeros((HIST_PAD - HIST,), jnp.float32)])
    out = _sc_forward(uidx8, cidx8, iidx64, hidx, gh, rat, noi,
                      preds.astype(jnp.float32), user_table, recipe_table,
                      category_table)
    return out.reshape(1, OUT_LEN)
